# speedup vs baseline: 29.6724x; 29.6724x over previous
import functools

import jax
import jax.numpy as jnp
import numpy as np
from jax.experimental import pallas as pl
from jax.experimental.pallas import tpu as pltpu

_C = 2048
_FOLD_DIV = 8


def _shift_codes():
    rng = np.random.default_rng(0)
    perm = rng.permutation(_C)
    fold = _C // _FOLD_DIV
    codes = np.zeros((_C,), np.int32)
    codes[np.sort(perm[:fold])] = 1
    codes[np.sort(perm[fold:2 * fold])] = 2
    return codes


def _tc_body(code_ref, x_ref, o_ref):
    xb = x_ref[0]
    lane = jax.lax.broadcasted_iota(jnp.int32, xb.shape, 1)
    left = jnp.roll(xb, -1, axis=1)
    left = jnp.where(lane == xb.shape[1] - 1, 0.0, left)
    right = jnp.roll(xb, 1, axis=1)
    right = jnp.where(lane == 0, 0.0, right)
    code = code_ref[...]
    o_ref[0] = jnp.where(code == 1, left, jnp.where(code == 2, right, xb))


@jax.jit
def _tc_shift(x, codes):
    B, C, T = x.shape
    return pl.pallas_call(
        _tc_body,
        grid=(B,),
        in_specs=[
            pl.BlockSpec((C, 1), lambda b: (0, 0)),
            pl.BlockSpec((1, C, T), lambda b: (b, 0, 0)),
        ],
        out_specs=pl.BlockSpec((1, C, T), lambda b: (b, 0, 0)),
        out_shape=jax.ShapeDtypeStruct((B, C, T), x.dtype),
        compiler_params=pltpu.CompilerParams(
            dimension_semantics=("arbitrary",)),
    )(codes, x)


def kernel(x):
    codes = jnp.asarray(_shift_codes()).reshape(_C, 1)
    return _tc_shift(x, codes)


# speedup vs baseline: 36.4277x; 1.2277x over previous
import functools

import jax
import jax.numpy as jnp
import numpy as np
from jax import lax
from jax.experimental import pallas as pl
from jax.experimental.pallas import tpu as pltpu
from jax.experimental.pallas import tpu_sc as plsc

_B = 128
_C = 2048
_T = 128
_P = _C // 2
_FOLD_DIV = 8
_NC = 2
_NS = 16
_NW = _NC * _NS
_K = 64
_RI = 7
_LEAD = 5


def _shift_codes():
    rng = np.random.default_rng(0)
    perm = rng.permutation(_C)
    fold = _C // _FOLD_DIV
    codes = np.zeros((_C,), np.int32)
    codes[np.sort(perm[:fold])] = 1
    codes[np.sort(perm[fold:2 * fold])] = 2
    return codes


def _build_chunks():
    codes = _shift_codes().reshape(_P, 2)
    types = {}
    for p in range(_P):
        t = (int(codes[p, 0]), int(codes[p, 1]))
        types.setdefault(t, []).append(p)
    rows = np.arange(_B * _P, dtype=np.int32).reshape(_B, _P)
    idx_parts = []
    chunk_types = []
    for t, plist in sorted(types.items()):
        r = rows[:, np.asarray(plist, np.int32)].reshape(-1)
        pad = (-r.size) % _NW
        if pad:
            r = np.concatenate([r, np.repeat(r[-1:], pad)])
        r = r.reshape(_NW, -1)
        n_w = r.shape[1]
        nch = -(-n_w // _K)
        pad2 = nch * _K - n_w
        if pad2:
            r = np.concatenate([r, np.repeat(r[:, -1:], pad2, axis=1)], axis=1)
        idx_parts.append(r.reshape(_NW, nch, _K))
        chunk_types += [t] * nch
    return np.concatenate(idx_parts, axis=1), chunk_types


_IDX, _CHUNK_TYPES = _build_chunks()
_NCH = _IDX.shape[1]

_CLEAN = [k for k, t in enumerate(_CHUNK_TYPES) if t == (0, 0)]
_PROC = [k for k, t in enumerate(_CHUNK_TYPES) if t != (0, 0)]
_ORDER = []
_ci = _pi = 0
while _ci < len(_CLEAN) or _pi < len(_PROC):
    if _ci < len(_CLEAN):
        _ORDER.append(_CLEAN[_ci])
        _ci += 1
    if _pi < len(_PROC):
        _ORDER.append(_PROC[_pi])
        _pi += 1


def _lane_perm(v, idx):
    return lax.gather(
        v, idx[:, None],
        dimension_numbers=lax.GatherDimensionNumbers(
            offset_dims=(), collapsed_slice_dims=(0,), start_index_map=(0,)),
        slice_sizes=(1,),
        mode=lax.GatherScatterMode.PROMISE_IN_BOUNDS)


def _process_chunk(buf, pair_type):
    lane = lax.iota(jnp.int32, 16)
    idx_l = jnp.minimum(lane + 1, 15)
    idx_r = jnp.maximum(lane - 1, 0)

    def half(r, h, code):
        if code == 1:
            for i in range(7):
                buf[r, h, pl.ds(i * 16, 16)] = buf[r, h, pl.ds(i * 16 + 1, 16)]
            v = _lane_perm(buf[r, h, pl.ds(112, 16)], idx_l)
            v = jnp.where(lane == 15, 0.0, v)
            buf[r, h, pl.ds(112, 16)] = v
        elif code == 2:
            for i in range(7, 0, -1):
                buf[r, h, pl.ds(i * 16, 16)] = buf[r, h, pl.ds(i * 16 - 1, 16)]
            v = _lane_perm(buf[r, h, pl.ds(0, 16)], idx_r)
            v = jnp.where(lane == 0, 0.0, v)
            buf[r, h, pl.ds(0, 16)] = v

    def body(r, carry):
        half(r, 0, pair_type[0])
        half(r, 1, pair_type[1])
        return carry

    lax.fori_loop(0, _K, body, 0)


_mesh = plsc.VectorSubcoreMesh(
    core_axis_name="c", subcore_axis_name="s",
    num_cores=_NC, num_subcores=_NS)


@functools.partial(
    pl.kernel,
    out_type=jax.ShapeDtypeStruct((_B * _P, 2, _T), jnp.float32),
    mesh=_mesh,
    scratch_types=[
        pltpu.VMEM((_NCH, _K), jnp.int32),
        [pltpu.VMEM((_K, 2, _T), jnp.float32) for _ in range(_RI)],
        [pltpu.SemaphoreType.DMA for _ in range(_RI)],
        [pltpu.SemaphoreType.DMA for _ in range(_RI)],
    ],
)
def _sc_shift(x_hbm, gidx_hbm, o_hbm, vidx, in_bufs, gsems, ssems):
    wid = lax.axis_index("s") * _NC + lax.axis_index("c")
    pltpu.sync_copy(gidx_hbm.at[wid], vidx)

    n = len(_ORDER)
    gh = {}
    in_pending = [None] * _RI

    def start_gather(i):
        k = _ORDER[i]
        s = i % _RI
        if in_pending[s] is not None:
            in_pending[s].wait()
            in_pending[s] = None
        gh[i] = pltpu.async_copy(
            x_hbm.at[vidx.at[k]], in_bufs[s], gsems[s])

    for i in range(min(_LEAD, n)):
        start_gather(i)
    for i in range(n):
        k = _ORDER[i]
        s = i % _RI
        gh.pop(i).wait()
        if _CHUNK_TYPES[k] != (0, 0):
            _process_chunk(in_bufs[s], _CHUNK_TYPES[k])
        in_pending[s] = pltpu.async_copy(
            in_bufs[s], o_hbm.at[vidx.at[k]], ssems[s])
        if i + _LEAD < n:
            start_gather(i + _LEAD)
    for h in in_pending:
        if h is not None:
            h.wait()


@jax.jit
def _run(x3, gidx):
    return _sc_shift(x3, gidx)


def kernel(x):
    B, C, T = x.shape
    out = _run(x.reshape(B * _P, 2, T), jnp.asarray(_IDX))
    return out.reshape(B, C, T)


# speedup vs baseline: 47.5692x; 1.3059x over previous
import functools

import jax
import jax.numpy as jnp
import numpy as np
from jax import lax
from jax.experimental import pallas as pl
from jax.experimental.pallas import tpu as pltpu
from jax.experimental.pallas import tpu_sc as plsc

_B = 128
_C = 2048
_T = 128
_FOLD_DIV = 8
_NC = 2
_NS = 16
_NW = _NC * _NS
_K = 128
_RI = 7
_LEAD = 5


def _shift_codes():
    rng = np.random.default_rng(0)
    perm = rng.permutation(_C)
    fold = _C // _FOLD_DIV
    codes = np.zeros((_C,), np.int32)
    codes[np.sort(perm[:fold])] = 1
    codes[np.sort(perm[fold:2 * fold])] = 2
    return codes


def _row_groups():
    codes = _shift_codes()
    rows = np.arange(_B * _C, dtype=np.int32).reshape(_B, _C)
    out = []
    for code in (1, 2, 0):
        ch = np.nonzero(codes == code)[0]
        r = rows[:, ch].reshape(-1)
        n = r.size
        assert n % (_NW * _K) == 0, (code, n)
        out.append(r.reshape(_NW, n // (_NW * _K), _K))
    return out


_IDX_FWD, _IDX_BWD, _IDX_FIX = _row_groups()
_NCH_F = _IDX_FWD.shape[1]
_NCH_B = _IDX_BWD.shape[1]
_NCH_X = _IDX_FIX.shape[1]


def _chunk_order():
    shifted = [("f", j, 1) for j in range(_NCH_F)]
    shifted += [("b", j, 2) for j in range(_NCH_B)]
    fixed = [("x", j, 0) for j in range(_NCH_X)]
    order = []
    fi = si = 0
    while fi < len(fixed) or si < len(shifted):
        for _ in range(3):
            if fi < len(fixed):
                order.append(fixed[fi])
                fi += 1
        if si < len(shifted):
            order.append(shifted[si])
            si += 1
    return order


_CHUNKS = _chunk_order()


def _lane_perm(v, idx):
    return lax.gather(
        v, idx[:, None],
        dimension_numbers=lax.GatherDimensionNumbers(
            offset_dims=(), collapsed_slice_dims=(0,), start_index_map=(0,)),
        slice_sizes=(1,),
        mode=lax.GatherScatterMode.PROMISE_IN_BOUNDS)


def _shift_chunk(buf, code):
    lane = lax.iota(jnp.int32, 16)
    if code == 1:
        bidx = jnp.minimum(lane + 1, 15)
    else:
        bidx = jnp.maximum(lane - 1, 0)

    def body(r, carry):
        if code == 1:
            for i in range(7):
                buf[r, pl.ds(i * 16, 16)] = buf[r, pl.ds(i * 16 + 1, 16)]
            v = _lane_perm(buf[r, pl.ds(112, 16)], bidx)
            v = jnp.where(lane == 15, 0.0, v)
            buf[r, pl.ds(112, 16)] = v
        else:
            for i in range(7, 0, -1):
                buf[r, pl.ds(i * 16, 16)] = buf[r, pl.ds(i * 16 - 1, 16)]
            v = _lane_perm(buf[r, pl.ds(0, 16)], bidx)
            v = jnp.where(lane == 0, 0.0, v)
            buf[r, pl.ds(0, 16)] = v
        return carry

    lax.fori_loop(0, _K, body, 0)


_mesh = plsc.VectorSubcoreMesh(
    core_axis_name="c", subcore_axis_name="s",
    num_cores=_NC, num_subcores=_NS)


@functools.partial(
    pl.kernel,
    out_type=jax.ShapeDtypeStruct((_B * _C, _T), jnp.float32),
    mesh=_mesh,
    scratch_types=[
        pltpu.VMEM((_NCH_F, _K), jnp.int32),
        pltpu.VMEM((_NCH_B, _K), jnp.int32),
        pltpu.VMEM((_NCH_X, _K), jnp.int32),
        [pltpu.VMEM((_K, _T), jnp.float32) for _ in range(_RI)],
        [pltpu.SemaphoreType.DMA for _ in range(_RI)],
        [pltpu.SemaphoreType.DMA for _ in range(_RI)],
    ],
)
def _sc_shift(x_hbm, gf_hbm, gb_hbm, gx_hbm, o_hbm,
              vf, vb, vx, in_bufs, gsems, ssems):
    wid = lax.axis_index("s") * _NC + lax.axis_index("c")
    pltpu.sync_copy(gf_hbm.at[wid], vf)
    pltpu.sync_copy(gb_hbm.at[wid], vb)
    pltpu.sync_copy(gx_hbm.at[wid], vx)

    idx_refs = {"f": vf, "b": vb, "x": vx}
    n = len(_CHUNKS)
    gh = {}
    in_pending = [None] * _RI

    def start_gather(k):
        g, j, _ = _CHUNKS[k]
        s = k % _RI
        if in_pending[s] is not None:
            in_pending[s].wait()
            in_pending[s] = None
        gh[k] = pltpu.async_copy(
            x_hbm.at[idx_refs[g].at[j]], in_bufs[s], gsems[s])

    for k in range(min(_LEAD, n)):
        start_gather(k)
    for k in range(n):
        g, j, code = _CHUNKS[k]
        s = k % _RI
        gh.pop(k).wait()
        if code != 0:
            _shift_chunk(in_bufs[s], code)
        in_pending[s] = pltpu.async_copy(
            in_bufs[s], o_hbm.at[idx_refs[g].at[j]], ssems[s])
        if k + _LEAD < n:
            start_gather(k + _LEAD)
    for h in in_pending:
        if h is not None:
            h.wait()


@jax.jit
def _run(x2, gf, gb, gx):
    return _sc_shift(x2, gf, gb, gx)


def kernel(x):
    B, C, T = x.shape
    out = _run(x.reshape(B * C, T),
               jnp.asarray(_IDX_FWD), jnp.asarray(_IDX_BWD),
               jnp.asarray(_IDX_FIX))
    return out.reshape(B, C, T)
